# baseline (speedup 1.0000x reference)
import functools

import jax
import jax.numpy as jnp
from jax import lax
from jax.experimental import pallas as pl
from jax.experimental.pallas import tpu as pltpu
from jax.experimental.pallas import tpu_sc as plsc

N = 10000
E = 320000
D = 128
NS = 16
C = 128
G = 2
NCH = 160
NGRP = NCH // G
EPT = NCH * C
EP = EPT * NS
NP = 10240
DSL = NP // NS
RSL = NP // NS

def _deg_body(dst4, ones_hbm, out_ref, acc_sh, idx_v, ones_v, s):
    slab = pl.ds(s * DSL, DSL)
    pltpu.sync_copy(ones_hbm.at[slab], acc_sh.at[slab])
    pltpu.sync_copy(ones_hbm.at[pl.ds(0, C)], ones_v)
    plsc.subcore_barrier()

    def body(j, carry):
        for g in range(G):
            pltpu.sync_copy(dst4.at[s, j, g], idx_v)
            pltpu.sync_copy(ones_v, acc_sh.at[idx_v], add=True)
        return carry

    lax.fori_loop(0, NGRP, body, 0)
    plsc.subcore_barrier()
    pltpu.sync_copy(acc_sh.at[slab], out_ref.at[slab])


def _prop_body(src4, dst4, u_hbm, out_ref, acc_sh,
               sidx_a, didx_a, sidx_b, didx_b, rows_a, rows_b,
               sem_a, sem_b, s):
    slab = pl.ds(s * RSL, RSL)
    pltpu.sync_copy(u_hbm.at[slab], acc_sh.at[slab])
    plsc.subcore_barrier()

    def body(j, carry):
        pltpu.sync_copy(src4.at[s, j, 0], sidx_a)
        pltpu.sync_copy(dst4.at[s, j, 0], didx_a)
        da = pltpu.async_copy(u_hbm.at[sidx_a], rows_a, sem_a)
        pltpu.sync_copy(src4.at[s, j, 1], sidx_b)
        pltpu.sync_copy(dst4.at[s, j, 1], didx_b)
        db = pltpu.async_copy(u_hbm.at[sidx_b], rows_b, sem_b)
        da.wait()
        pltpu.sync_copy(rows_a, acc_sh.at[didx_a], add=True)
        db.wait()
        pltpu.sync_copy(rows_b, acc_sh.at[didx_b], add=True)
        return carry

    lax.fori_loop(0, NGRP, body, 0)
    plsc.subcore_barrier()
    pltpu.sync_copy(acc_sh.at[slab], out_ref.at[slab])


@functools.cache
def _sc_kernels():
    mesh = plsc.VectorSubcoreMesh(core_axis_name="c", subcore_axis_name="s",
                                  num_cores=2, num_subcores=NS)

    @functools.partial(
        pl.kernel,
        out_type=(
            jax.ShapeDtypeStruct((NP, 1), jnp.float32),
            jax.ShapeDtypeStruct((NP, 1), jnp.float32),
        ),
        mesh=mesh,
        scratch_types=[
            pltpu.VMEM_SHARED((NP, 1), jnp.float32),
            pltpu.VMEM((C,), jnp.int32),
            pltpu.VMEM((C, 1), jnp.float32),
        ],
    )
    def deg_kernel(dst_ex, dst_im, ones_hbm, out_ex, out_im,
                   acc_sh, idx_v, ones_v):
        c = lax.axis_index("c")
        s = lax.axis_index("s")

        @pl.when(c == 0)
        def _():
            _deg_body(dst_ex, ones_hbm, out_ex, acc_sh, idx_v, ones_v, s)

        @pl.when(c == 1)
        def _():
            _deg_body(dst_im, ones_hbm, out_im, acc_sh, idx_v, ones_v, s)

    @functools.partial(
        pl.kernel,
        out_type=(
            jax.ShapeDtypeStruct((NP, D), jnp.float32),
            jax.ShapeDtypeStruct((NP, D), jnp.float32),
        ),
        mesh=mesh,
        scratch_types=[
            pltpu.VMEM_SHARED((NP, D), jnp.float32),
            pltpu.VMEM((C,), jnp.int32),
            pltpu.VMEM((C,), jnp.int32),
            pltpu.VMEM((C,), jnp.int32),
            pltpu.VMEM((C,), jnp.int32),
            pltpu.VMEM((C, D), jnp.float32),
            pltpu.VMEM((C, D), jnp.float32),
            pltpu.SemaphoreType.DMA,
            pltpu.SemaphoreType.DMA,
        ],
    )
    def prop_kernel(src_ex, dst_ex, src_im, dst_im, u_ex, u_im,
                    out_ex, out_im, acc_sh, sidx_a, didx_a, sidx_b, didx_b,
                    rows_a, rows_b, sem_a, sem_b):
        c = lax.axis_index("c")
        s = lax.axis_index("s")

        @pl.when(c == 0)
        def _():
            _prop_body(src_ex, dst_ex, u_ex, out_ex, acc_sh, sidx_a, didx_a,
                       sidx_b, didx_b, rows_a, rows_b, sem_a, sem_b, s)

        @pl.when(c == 1)
        def _():
            _prop_body(src_im, dst_im, u_im, out_im, acc_sh, sidx_a, didx_a,
                       sidx_b, didx_b, rows_a, rows_b, sem_a, sem_b, s)

    return deg_kernel, prop_kernel


R = 1024


def _tc1_body(deg_e, deg_i, xe, xi, we, wi, dbe_o, dbi_o, ue_o, ui_o):
    dbe = jnp.broadcast_to(lax.rsqrt(deg_e[...]), (R, D))
    dbi = jnp.broadcast_to(lax.rsqrt(deg_i[...]), (R, D))
    dbe_o[...] = dbe
    dbi_o[...] = dbi
    ue_o[...] = dbe * jnp.dot(xe[...], we[...], preferred_element_type=jnp.float32)
    ui_o[...] = dbi * jnp.dot(xi[...], wi[...], preferred_element_type=jnp.float32)


def _tc1(deg_e, deg_i, emb_ex, emb_im, W_ex1, W_im1):
    nd = jax.ShapeDtypeStruct((NP, D), jnp.float32)
    col = pl.BlockSpec((R, 1), lambda i: (i, 0))
    row = pl.BlockSpec((R, D), lambda i: (i, 0))
    w = pl.BlockSpec((D, D), lambda i: (0, 0))
    return pl.pallas_call(
        _tc1_body,
        grid=(NP // R,),
        in_specs=[col, col, row, row, w, w],
        out_specs=[row, row, row, row],
        out_shape=[nd, nd, nd, nd],
    )(deg_e, deg_i, emb_ex, emb_im, W_ex1, W_im1)


def _sigmoid(z):
    return 1.0 / (1.0 + jnp.exp(-z))


def _tc2_body(spu_e, spu_i, dbe_r, dbi_r, be1, bi1,
              tm1w, tm1b, tm2w, tm2b, g1w, g1b, g2w, g2b, we2, wi2,
              u2e_o, u2i_o, acc1_o):
    f32 = jnp.float32
    dbe = dbe_r[...]
    dbi = dbi_r[...]
    x1e = dbe * spu_e[...] + be1[...]
    x1i = dbi * spu_i[...] + bi1[...]
    e2t = jnp.dot(x1i, tm1w[...], preferred_element_type=f32) + tm1b[...]
    e1t = jnp.dot(x1e, tm2w[...], preferred_element_type=f32) + tm2b[...]
    g1wv = g1w[...]
    g2wv = g2w[...]
    g1 = _sigmoid(jnp.dot(x1e, g1wv[:D, :], preferred_element_type=f32)
                  + jnp.dot(e2t, g1wv[D:, :], preferred_element_type=f32)
                  + g1b[...])
    g2 = _sigmoid(jnp.dot(x1i, g2wv[:D, :], preferred_element_type=f32)
                  + jnp.dot(e1t, g2wv[D:, :], preferred_element_type=f32)
                  + g2b[...])
    te = g1 * e2t + x1e
    ti = g2 * e1t + x1i
    u2e_o[...] = dbe * jnp.dot(te, we2[...], preferred_element_type=f32)
    u2i_o[...] = dbi * jnp.dot(ti, wi2[...], preferred_element_type=f32)
    acc1_o[...] = x1e + x1i


def _tc2(spu_e, spu_i, dbe, dbi, b_ex1, b_im1,
         tm1_W, tm1_b, tm2_W, tm2_b, g1_W, g1_b, g2_W, g2_b, W_ex2, W_im2):
    nd = jax.ShapeDtypeStruct((NP, D), jnp.float32)
    row = pl.BlockSpec((R, D), lambda i: (i, 0))
    w = pl.BlockSpec((D, D), lambda i: (0, 0))
    w2 = pl.BlockSpec((2 * D, D), lambda i: (0, 0))
    b = pl.BlockSpec((1, D), lambda i: (0, 0))
    return pl.pallas_call(
        _tc2_body,
        grid=(NP // R,),
        in_specs=[row, row, row, row, b, b, w, b, w, b, w2, b, w2, b, w, w],
        out_specs=[row, row, row],
        out_shape=[nd, nd, nd],
    )(spu_e, spu_i, dbe, dbi, b_ex1.reshape(1, D), b_im1.reshape(1, D),
      tm1_W, tm1_b.reshape(1, D), tm2_W, tm2_b.reshape(1, D),
      g1_W, g1_b.reshape(1, D), g2_W, g2_b.reshape(1, D), W_ex2, W_im2)


def _tc3_body(acc1, spu2_e, spu2_i, dbe, dbi, be2, bi2, out_o):
    out_o[...] = (acc1[...]
                  + dbe[...] * spu2_e[...] + be2[...]
                  + dbi[...] * spu2_i[...] + bi2[...])


def _tc3(acc1, spu2_e, spu2_i, dbe, dbi, b_ex2, b_im2):
    nd = jax.ShapeDtypeStruct((NP, D), jnp.float32)
    row = pl.BlockSpec((R, D), lambda i: (i, 0))
    b = pl.BlockSpec((1, D), lambda i: (0, 0))
    return pl.pallas_call(
        _tc3_body,
        grid=(NP // R,),
        in_specs=[row, row, row, row, row, b, b],
        out_specs=row,
        out_shape=nd,
    )(acc1, spu2_e, spu2_i, dbe, dbi, b_ex2.reshape(1, D), b_im2.reshape(1, D))


def kernel(edge_index_ex, edge_type_ex, edge_index_im, edge_type_im,
           emb_ex, emb_im,
           W_ex1, W_ex2, W_im1, W_im2,
           b_ex1, b_ex2, b_im1, b_im2,
           tm1_W, tm1_b, tm2_W, tm2_b,
           g1_W, g1_b, g2_W, g2_b):
    deg_kernel, prop_kernel = _sc_kernels()

    def prep(ei):
        src = jnp.pad(ei[0], (0, EP - E)).reshape(NS, NGRP, G, C)
        dst = jnp.pad(ei[1], (0, EP - E),
                      constant_values=N).reshape(NS, NGRP, G, C)
        return src, dst

    src_ex, dst_ex = prep(edge_index_ex)
    src_im, dst_im = prep(edge_index_im)
    ones = jnp.ones((NP, 1), jnp.float32)
    pad = ((0, NP - N), (0, 0))
    emb_ex_p = jnp.pad(emb_ex, pad)
    emb_im_p = jnp.pad(emb_im, pad)
    deg_e, deg_i = deg_kernel(dst_ex, dst_im, ones)
    dbe, dbi, u1e, u1i = _tc1(deg_e, deg_i, emb_ex_p, emb_im_p, W_ex1, W_im1)
    spu1e, spu1i = prop_kernel(src_ex, dst_ex, src_im, dst_im, u1e, u1i)
    u2e, u2i, acc1 = _tc2(spu1e, spu1i, dbe, dbi, b_ex1, b_im1,
                          tm1_W, tm1_b, tm2_W, tm2_b,
                          g1_W, g1_b, g2_W, g2_b, W_ex2, W_im2)
    spu2e, spu2i = prop_kernel(src_ex, dst_ex, src_im, dst_im, u2e, u2i)
    return _tc3(acc1, spu2e, spu2i, dbe, dbi, b_ex2, b_im2)[:N]


# speedup vs baseline: 1.2483x; 1.2483x over previous
import functools

import jax
import jax.numpy as jnp
from jax import lax
from jax.experimental import pallas as pl
from jax.experimental.pallas import tpu as pltpu
from jax.experimental.pallas import tpu_sc as plsc

N = 10000
E = 320000
D = 128
NS = 16
C = 128
NCH = 158
NPAIR = NCH // 2
EPT = NCH * C
EP = EPT * NS
NP = 10240
DSL = NP // NS
RSL = NP // NS

def _deg_body(sd3, ones_hbm, out_ref, acc_sh, idx_v, ones_v, s):
    slab = pl.ds(s * DSL, DSL)
    pltpu.sync_copy(ones_hbm.at[slab], acc_sh.at[slab])
    pltpu.sync_copy(ones_hbm.at[pl.ds(0, C)], ones_v)
    plsc.subcore_barrier()

    def body(i, carry):
        pltpu.sync_copy(sd3.at[s, i, 1], idx_v)
        pltpu.sync_copy(ones_v, acc_sh.at[idx_v], add=True)
        return carry

    lax.fori_loop(0, NCH, body, 0)
    plsc.subcore_barrier()
    pltpu.sync_copy(acc_sh.at[slab], out_ref.at[slab])


def _prop_body(sd3, u_hbm, out_ref, acc_sh,
               sidx_a, didx_a, sidx_b, didx_b, rows_a, rows_b,
               sem_a, sem_b, s):
    slab = pl.ds(s * RSL, RSL)
    pltpu.sync_copy(u_hbm.at[slab], acc_sh.at[slab])
    plsc.subcore_barrier()

    def body(j, carry):
        i0 = 2 * j
        pltpu.sync_copy(sd3.at[s, i0, 0], sidx_a)
        pltpu.sync_copy(sd3.at[s, i0, 1], didx_a)
        da = pltpu.async_copy(u_hbm.at[sidx_a], rows_a, sem_a)
        pltpu.sync_copy(sd3.at[s, i0 + 1, 0], sidx_b)
        pltpu.sync_copy(sd3.at[s, i0 + 1, 1], didx_b)
        db = pltpu.async_copy(u_hbm.at[sidx_b], rows_b, sem_b)
        da.wait()
        pltpu.sync_copy(rows_a, acc_sh.at[didx_a], add=True)
        db.wait()
        pltpu.sync_copy(rows_b, acc_sh.at[didx_b], add=True)
        return carry

    lax.fori_loop(0, NPAIR, body, 0)
    plsc.subcore_barrier()
    pltpu.sync_copy(acc_sh.at[slab], out_ref.at[slab])


@functools.cache
def _sc_kernels():
    mesh = plsc.VectorSubcoreMesh(core_axis_name="c", subcore_axis_name="s",
                                  num_cores=2, num_subcores=NS)

    @functools.partial(
        pl.kernel,
        out_type=(
            jax.ShapeDtypeStruct((NP, 1), jnp.float32),
            jax.ShapeDtypeStruct((NP, 1), jnp.float32),
        ),
        mesh=mesh,
        scratch_types=[
            pltpu.VMEM_SHARED((NP, 1), jnp.float32),
            pltpu.VMEM((C,), jnp.int32),
            pltpu.VMEM((C, 1), jnp.float32),
        ],
    )
    def deg_kernel(sd_ex, sd_im, ones_hbm, out_ex, out_im,
                   acc_sh, idx_v, ones_v):
        c = lax.axis_index("c")
        s = lax.axis_index("s")

        @pl.when(c == 0)
        def _():
            _deg_body(sd_ex, ones_hbm, out_ex, acc_sh, idx_v, ones_v, s)

        @pl.when(c == 1)
        def _():
            _deg_body(sd_im, ones_hbm, out_im, acc_sh, idx_v, ones_v, s)

    @functools.partial(
        pl.kernel,
        out_type=(
            jax.ShapeDtypeStruct((NP, D), jnp.float32),
            jax.ShapeDtypeStruct((NP, D), jnp.float32),
        ),
        mesh=mesh,
        scratch_types=[
            pltpu.VMEM_SHARED((NP, D), jnp.float32),
            pltpu.VMEM((C,), jnp.int32),
            pltpu.VMEM((C,), jnp.int32),
            pltpu.VMEM((C,), jnp.int32),
            pltpu.VMEM((C,), jnp.int32),
            pltpu.VMEM((C, D), jnp.float32),
            pltpu.VMEM((C, D), jnp.float32),
            pltpu.SemaphoreType.DMA,
            pltpu.SemaphoreType.DMA,
        ],
    )
    def prop_kernel(sd_ex, sd_im, u_ex, u_im, out_ex, out_im,
                    acc_sh, sidx_a, didx_a, sidx_b, didx_b,
                    rows_a, rows_b, sem_a, sem_b):
        c = lax.axis_index("c")
        s = lax.axis_index("s")

        @pl.when(c == 0)
        def _():
            _prop_body(sd_ex, u_ex, out_ex, acc_sh, sidx_a, didx_a,
                       sidx_b, didx_b, rows_a, rows_b, sem_a, sem_b, s)

        @pl.when(c == 1)
        def _():
            _prop_body(sd_im, u_im, out_im, acc_sh, sidx_a, didx_a,
                       sidx_b, didx_b, rows_a, rows_b, sem_a, sem_b, s)

    return deg_kernel, prop_kernel


R = 1024


def _tc1_body(deg_e, deg_i, xe, xi, we, wi, dbe_o, dbi_o, ue_o, ui_o):
    dbe = jnp.broadcast_to(lax.rsqrt(deg_e[...]), (R, D))
    dbi = jnp.broadcast_to(lax.rsqrt(deg_i[...]), (R, D))
    dbe_o[...] = dbe
    dbi_o[...] = dbi
    ue_o[...] = dbe * jnp.dot(xe[...], we[...], preferred_element_type=jnp.float32)
    ui_o[...] = dbi * jnp.dot(xi[...], wi[...], preferred_element_type=jnp.float32)


def _tc1(deg_e, deg_i, emb_ex, emb_im, W_ex1, W_im1):
    nd = jax.ShapeDtypeStruct((NP, D), jnp.float32)
    col = pl.BlockSpec((R, 1), lambda i: (i, 0))
    row = pl.BlockSpec((R, D), lambda i: (i, 0))
    w = pl.BlockSpec((D, D), lambda i: (0, 0))
    return pl.pallas_call(
        _tc1_body,
        grid=(NP // R,),
        in_specs=[col, col, row, row, w, w],
        out_specs=[row, row, row, row],
        out_shape=[nd, nd, nd, nd],
    )(deg_e, deg_i, emb_ex, emb_im, W_ex1, W_im1)


def _sigmoid(z):
    return 1.0 / (1.0 + jnp.exp(-z))


def _tc2_body(spu_e, spu_i, dbe_r, dbi_r, be1, bi1,
              tm1w, tm1b, tm2w, tm2b, g1w, g1b, g2w, g2b, we2, wi2,
              u2e_o, u2i_o, acc1_o):
    f32 = jnp.float32
    dbe = dbe_r[...]
    dbi = dbi_r[...]
    x1e = dbe * spu_e[...] + be1[...]
    x1i = dbi * spu_i[...] + bi1[...]
    e2t = jnp.dot(x1i, tm1w[...], preferred_element_type=f32) + tm1b[...]
    e1t = jnp.dot(x1e, tm2w[...], preferred_element_type=f32) + tm2b[...]
    g1wv = g1w[...]
    g2wv = g2w[...]
    g1 = _sigmoid(jnp.dot(x1e, g1wv[:D, :], preferred_element_type=f32)
                  + jnp.dot(e2t, g1wv[D:, :], preferred_element_type=f32)
                  + g1b[...])
    g2 = _sigmoid(jnp.dot(x1i, g2wv[:D, :], preferred_element_type=f32)
                  + jnp.dot(e1t, g2wv[D:, :], preferred_element_type=f32)
                  + g2b[...])
    te = g1 * e2t + x1e
    ti = g2 * e1t + x1i
    u2e_o[...] = dbe * jnp.dot(te, we2[...], preferred_element_type=f32)
    u2i_o[...] = dbi * jnp.dot(ti, wi2[...], preferred_element_type=f32)
    acc1_o[...] = x1e + x1i


def _tc2(spu_e, spu_i, dbe, dbi, b_ex1, b_im1,
         tm1_W, tm1_b, tm2_W, tm2_b, g1_W, g1_b, g2_W, g2_b, W_ex2, W_im2):
    nd = jax.ShapeDtypeStruct((NP, D), jnp.float32)
    row = pl.BlockSpec((R, D), lambda i: (i, 0))
    w = pl.BlockSpec((D, D), lambda i: (0, 0))
    w2 = pl.BlockSpec((2 * D, D), lambda i: (0, 0))
    b = pl.BlockSpec((1, D), lambda i: (0, 0))
    return pl.pallas_call(
        _tc2_body,
        grid=(NP // R,),
        in_specs=[row, row, row, row, b, b, w, b, w, b, w2, b, w2, b, w, w],
        out_specs=[row, row, row],
        out_shape=[nd, nd, nd],
    )(spu_e, spu_i, dbe, dbi, b_ex1.reshape(1, D), b_im1.reshape(1, D),
      tm1_W, tm1_b.reshape(1, D), tm2_W, tm2_b.reshape(1, D),
      g1_W, g1_b.reshape(1, D), g2_W, g2_b.reshape(1, D), W_ex2, W_im2)


def _tc3_body(acc1, spu2_e, spu2_i, dbe, dbi, be2, bi2, out_o):
    out_o[...] = (acc1[...]
                  + dbe[...] * spu2_e[...] + be2[...]
                  + dbi[...] * spu2_i[...] + bi2[...])


def _tc3(acc1, spu2_e, spu2_i, dbe, dbi, b_ex2, b_im2):
    nd = jax.ShapeDtypeStruct((NP, D), jnp.float32)
    row = pl.BlockSpec((R, D), lambda i: (i, 0))
    b = pl.BlockSpec((1, D), lambda i: (0, 0))
    return pl.pallas_call(
        _tc3_body,
        grid=(NP // R,),
        in_specs=[row, row, row, row, row, b, b],
        out_specs=row,
        out_shape=nd,
    )(acc1, spu2_e, spu2_i, dbe, dbi, b_ex2.reshape(1, D), b_im2.reshape(1, D))


def kernel(edge_index_ex, edge_type_ex, edge_index_im, edge_type_im,
           emb_ex, emb_im,
           W_ex1, W_ex2, W_im1, W_im2,
           b_ex1, b_ex2, b_im1, b_im2,
           tm1_W, tm1_b, tm2_W, tm2_b,
           g1_W, g1_b, g2_W, g2_b):
    deg_kernel, prop_kernel = _sc_kernels()

    def prep(ei):
        src = jnp.pad(ei[0], (0, EP - E)).reshape(NS, NCH, C)
        dst = jnp.pad(ei[1], (0, EP - E), constant_values=N).reshape(NS, NCH, C)
        return jnp.stack([src, dst], axis=2)

    sd_ex = prep(edge_index_ex)
    sd_im = prep(edge_index_im)
    ones = jnp.ones((NP, 1), jnp.float32)
    pad = ((0, NP - N), (0, 0))
    emb_ex_p = jnp.pad(emb_ex, pad)
    emb_im_p = jnp.pad(emb_im, pad)
    deg_e, deg_i = deg_kernel(sd_ex, sd_im, ones)
    dbe, dbi, u1e, u1i = _tc1(deg_e, deg_i, emb_ex_p, emb_im_p, W_ex1, W_im1)
    spu1e, spu1i = prop_kernel(sd_ex, sd_im, u1e, u1i)
    u2e, u2i, acc1 = _tc2(spu1e, spu1i, dbe, dbi, b_ex1, b_im1,
                          tm1_W, tm1_b, tm2_W, tm2_b,
                          g1_W, g1_b, g2_W, g2_b, W_ex2, W_im2)
    spu2e, spu2i = prop_kernel(sd_ex, sd_im, u2e, u2i)
    return _tc3(acc1, spu2e, spu2i, dbe, dbi, b_ex2, b_im2)[:N]
